# baseline (speedup 1.0000x reference)
import functools

import jax
import jax.numpy as jnp
from jax import lax
from jax.experimental import pallas as pl
from jax.experimental.pallas import tpu as pltpu
from jax.experimental.pallas import tpu_sc as plsc

K_NB = 16
R2 = 0.2 * 0.2
LANES = 16
NUM_CORES = 2
NUM_SUBCORES = 16
NUM_TILES = NUM_CORES * NUM_SUBCORES


def _build_sc_kernel(B, N, C, interpret=False):
    assert (B * N) % NUM_TILES == 0 and N % LANES == 0
    q_per_tile = (B * N) // NUM_TILES
    tiles_per_batch = NUM_TILES // B
    nchunk = N // LANES
    FIRST = 32
    GROUP = 16
    assert (nchunk - FIRST) % GROUP == 0
    ngroups = (nchunk - FIRST) // GROUP

    mesh = plsc.VectorSubcoreMesh(
        core_axis_name="c", subcore_axis_name="s",
        num_cores=NUM_CORES, num_subcores=NUM_SUBCORES)

    @functools.partial(
        pl.kernel,
        out_type=jax.ShapeDtypeStruct((NUM_TILES * LANES,), jnp.float32),
        mesh=mesh,
        interpret=interpret,
        compiler_params=pltpu.CompilerParams(
            use_tc_tiling_on_sc=False, needs_layout_passes=False),
        scratch_types=[
            pltpu.VMEM((3, N), jnp.float32),
            pltpu.VMEM((C * N,), jnp.float32),
            pltpu.VMEM((528,), jnp.int32),
            pltpu.VMEM((q_per_tile * LANES,), jnp.int32),
            pltpu.VMEM((LANES,), jnp.float32),
            pltpu.SemaphoreType.DMA,
        ],
    )
    def ballq(pc_hbm, mask_hbm, out_hbm, pc_v, mask_flat, idx_v, idxall_v,
              acc_v, sem):
        cid = lax.axis_index("c")
        sid = lax.axis_index("s")
        wid = sid * NUM_CORES + cid
        b = wid // tiles_per_batch
        qbase = (wid % tiles_per_batch) * q_per_tile

        mask_copy = pltpu.async_copy(mask_hbm.at[b], mask_flat, sem)
        pltpu.sync_copy(pc_hbm.at[b], pc_v)

        idx_v[pl.ds(0, LANES)] = jnp.zeros((LANES,), jnp.int32)

        iota = lax.iota(jnp.int32, LANES)

        def per_query(qi, total):
            q = qbase + qi
            qsplat = jnp.full((LANES,), q, jnp.int32)
            d0 = jnp.zeros((LANES,), jnp.int32)
            qx = plsc.load_gather(pc_v, [d0, qsplat])
            qy = plsc.load_gather(pc_v, [d0 + 1, qsplat])
            qz = plsc.load_gather(pc_v, [d0 + 2, qsplat])

            def scan_range(cbase, count, run_init):
                @plsc.parallel_loop(0, count, 1, unroll=min(count, GROUP),
                                    carry=run_init)
                def run_loop(j, run):
                    base = cbase + j * LANES
                    dx = pc_v[0, pl.ds(base, LANES)] - qx
                    dy = pc_v[1, pl.ds(base, LANES)] - qy
                    dz = pc_v[2, pl.ds(base, LANES)] - qz
                    d2 = dx * dx + dy * dy + dz * dz
                    m = d2 < R2
                    pos = run + plsc.cumsum(m.astype(jnp.int32)) - 1
                    plsc.store_scatter(idx_v, [pos], iota + base, mask=m)
                    return run + plsc.all_reduce_population_count(m)

                return run_loop

            p0 = scan_range(0, FIRST, jnp.zeros((LANES,), jnp.int32))[0]

            def scan_cond(carry):
                group, ptr = carry
                return jnp.logical_and(ptr < K_NB, group < ngroups)

            def scan_body(carry):
                group, ptr = carry
                run = scan_range(FIRST * LANES + group * (GROUP * LANES),
                                 GROUP, jnp.full((LANES,), ptr, jnp.int32))
                return group + 1, run[0]

            _, found = lax.while_loop(
                scan_cond, scan_body, (jnp.int32(0), p0))

            cnt16 = jnp.minimum(jnp.maximum(found, 1), K_NB)
            idxv = idx_v[pl.ds(0, LANES)]
            valid = iota < cnt16
            first = idxv[0]
            idxall_v[pl.ds(qi * LANES, LANES)] = jnp.where(valid, idxv, first)
            return total

        lax.fori_loop(0, q_per_tile, per_query, jnp.int32(0))
        mask_copy.wait()

        def per_query_gather(qi, total):
            q = qbase + qi
            idx_c = idxall_v[pl.ds(qi * LANES, LANES)]
            q_c = jnp.full((LANES,), q, jnp.int32)
            acc = jnp.zeros((LANES,), jnp.float32)
            for c in range(C):
                nm = plsc.load_gather(mask_flat, [idx_c])
                qm = plsc.load_gather(mask_flat, [q_c])
                acc = acc + jnp.abs(nm - qm)
                if c + 1 < C:
                    idx_c = idx_c + N
                    q_c = q_c + N
            return total + acc

        total = lax.fori_loop(0, q_per_tile, per_query_gather,
                              jnp.zeros((LANES,), jnp.float32))
        acc_v[...] = total
        pltpu.sync_copy(acc_v, out_hbm.at[pl.ds(wid * LANES, LANES)])

    return ballq


def kernel(pc, mask):
    B, N, _ = pc.shape
    C = mask.shape[-1]
    pcT = jnp.transpose(pc, (0, 2, 1))
    maskT = jnp.transpose(mask, (0, 2, 1)).reshape(B, C * N)
    partial = _build_sc_kernel(B, N, C)(pcT, maskT)
    return jnp.sum(partial) / (B * N * K_NB)


# speedup vs baseline: 1.0485x; 1.0485x over previous
import functools

import jax
import jax.numpy as jnp
from jax import lax
from jax.experimental import pallas as pl
from jax.experimental.pallas import tpu as pltpu
from jax.experimental.pallas import tpu_sc as plsc

K_NB = 16
R2 = 0.2 * 0.2
LANES = 16
NUM_CORES = 2
NUM_SUBCORES = 16
NUM_TILES = NUM_CORES * NUM_SUBCORES


def _build_sc_kernel(B, N, C, interpret=False):
    assert (B * N) % NUM_TILES == 0 and N % LANES == 0
    q_per_tile = (B * N) // NUM_TILES
    tiles_per_batch = NUM_TILES // B
    nchunk = N // LANES
    GROUP = 16
    assert nchunk % GROUP == 0
    ngroups = nchunk // GROUP

    mesh = plsc.VectorSubcoreMesh(
        core_axis_name="c", subcore_axis_name="s",
        num_cores=NUM_CORES, num_subcores=NUM_SUBCORES)

    @functools.partial(
        pl.kernel,
        out_type=jax.ShapeDtypeStruct((NUM_TILES * LANES,), jnp.float32),
        mesh=mesh,
        interpret=interpret,
        compiler_params=pltpu.CompilerParams(
            use_tc_tiling_on_sc=False, needs_layout_passes=False),
        scratch_types=[
            pltpu.VMEM((3, N), jnp.float32),
            pltpu.VMEM((C * N,), jnp.float32),
            pltpu.VMEM((304,), jnp.int32),
            pltpu.VMEM((q_per_tile * LANES,), jnp.int32),
            pltpu.VMEM((LANES,), jnp.float32),
            pltpu.SemaphoreType.DMA,
        ],
    )
    def ballq(pc_hbm, mask_hbm, out_hbm, pc_v, mask_flat, idx_v, idxall_v,
              acc_v, sem):
        cid = lax.axis_index("c")
        sid = lax.axis_index("s")
        wid = sid * NUM_CORES + cid
        b = wid // tiles_per_batch
        qbase = (wid % tiles_per_batch) * q_per_tile

        mask_copy = pltpu.async_copy(mask_hbm.at[b], mask_flat, sem)
        pltpu.sync_copy(pc_hbm.at[b], pc_v)

        idx_v[pl.ds(0, LANES)] = jnp.zeros((LANES,), jnp.int32)

        iota = lax.iota(jnp.int32, LANES)

        def per_query(qi, total):
            q = qbase + qi
            qsplat = jnp.full((LANES,), q, jnp.int32)
            d0 = jnp.zeros((LANES,), jnp.int32)
            qx = plsc.load_gather(pc_v, [d0, qsplat])
            qy = plsc.load_gather(pc_v, [d0 + 1, qsplat])
            qz = plsc.load_gather(pc_v, [d0 + 2, qsplat])

            def scan_cond(carry):
                group, ptr = carry
                return jnp.logical_and(ptr < K_NB, group < ngroups)

            def scan_body(carry):
                group, ptr = carry
                gbase = group * (GROUP * LANES)
                ones = jnp.ones((LANES,), jnp.int32)

                @plsc.parallel_loop(0, GROUP, 1, unroll=GROUP,
                                    carry=jnp.full((LANES,), ptr - 1, jnp.int32))
                def run_loop(j, run_m1):
                    base = gbase + j * LANES
                    dx = pc_v[0, pl.ds(base, LANES)] - qx
                    dy = pc_v[1, pl.ds(base, LANES)] - qy
                    dz = pc_v[2, pl.ds(base, LANES)] - qz
                    d2 = dx * dx + dy * dy + dz * dz
                    m = d2 < R2
                    pos = run_m1 + plsc.cumsum(ones, mask=m)
                    plsc.store_scatter(idx_v, [pos], iota + base, mask=m)
                    return run_m1 + plsc.all_reduce_population_count(m)

                return group + 1, run_loop[0] + 1

            _, found = lax.while_loop(
                scan_cond, scan_body, (jnp.int32(0), jnp.int32(0)))

            cnt16 = jnp.minimum(jnp.maximum(found, 1), K_NB)
            idxv = idx_v[pl.ds(0, LANES)]
            valid = iota < cnt16
            first = idxv[0]
            idxall_v[pl.ds(qi * LANES, LANES)] = jnp.where(valid, idxv, first)
            return total

        lax.fori_loop(0, q_per_tile, per_query, jnp.int32(0))
        mask_copy.wait()

        def per_query_gather(qi, total):
            q = qbase + qi
            idx_c = idxall_v[pl.ds(qi * LANES, LANES)]
            q_c = jnp.full((LANES,), q, jnp.int32)
            acc = jnp.zeros((LANES,), jnp.float32)
            for c in range(C):
                nm = plsc.load_gather(mask_flat, [idx_c])
                qm = plsc.load_gather(mask_flat, [q_c])
                acc = acc + jnp.abs(nm - qm)
                if c + 1 < C:
                    idx_c = idx_c + N
                    q_c = q_c + N
            return total + acc

        total = lax.fori_loop(0, q_per_tile, per_query_gather,
                              jnp.zeros((LANES,), jnp.float32))
        acc_v[...] = total
        pltpu.sync_copy(acc_v, out_hbm.at[pl.ds(wid * LANES, LANES)])

    return ballq


def kernel(pc, mask):
    B, N, _ = pc.shape
    C = mask.shape[-1]
    pcT = jnp.transpose(pc, (0, 2, 1))
    maskT = jnp.transpose(mask, (0, 2, 1)).reshape(B, C * N)
    partial = _build_sc_kernel(B, N, C)(pcT, maskT)
    return jnp.sum(partial) / (B * N * K_NB)


# speedup vs baseline: 1.0524x; 1.0038x over previous
import functools

import jax
import jax.numpy as jnp
from jax import lax
from jax.experimental import pallas as pl
from jax.experimental.pallas import tpu as pltpu
from jax.experimental.pallas import tpu_sc as plsc

K_NB = 16
R2 = 0.2 * 0.2
LANES = 16
NUM_CORES = 2
NUM_SUBCORES = 16
NUM_TILES = NUM_CORES * NUM_SUBCORES


def _build_sc_kernel(B, N, C, interpret=False):
    assert (B * N) % NUM_TILES == 0 and N % LANES == 0
    q_per_tile = (B * N) // NUM_TILES
    tiles_per_batch = NUM_TILES // B
    nchunk = N // LANES
    GROUP = 16
    assert nchunk % GROUP == 0
    ngroups = nchunk // GROUP

    mesh = plsc.VectorSubcoreMesh(
        core_axis_name="c", subcore_axis_name="s",
        num_cores=NUM_CORES, num_subcores=NUM_SUBCORES)

    @functools.partial(
        pl.kernel,
        out_type=jax.ShapeDtypeStruct((NUM_TILES * LANES,), jnp.float32),
        mesh=mesh,
        interpret=interpret,
        compiler_params=pltpu.CompilerParams(
            use_tc_tiling_on_sc=False, needs_layout_passes=False),
        scratch_types=[
            pltpu.VMEM((3, N), jnp.float32),
            pltpu.VMEM((C * N,), jnp.float32),
            pltpu.VMEM((304,), jnp.int32),
            pltpu.VMEM((q_per_tile * LANES,), jnp.int32),
            pltpu.VMEM((LANES,), jnp.float32),
            pltpu.SemaphoreType.DMA,
        ],
    )
    def ballq(pc_hbm, mask_hbm, out_hbm, pc_v, mask_flat, idx_v, idxall_v,
              acc_v, sem):
        cid = lax.axis_index("c")
        sid = lax.axis_index("s")
        wid = sid * NUM_CORES + cid
        b = wid // tiles_per_batch
        qbase = (wid % tiles_per_batch) * q_per_tile

        mask_copy = pltpu.async_copy(mask_hbm.at[b], mask_flat, sem)
        pltpu.sync_copy(pc_hbm.at[b], pc_v)

        idx_v[pl.ds(0, LANES)] = jnp.zeros((LANES,), jnp.int32)

        iota = lax.iota(jnp.int32, LANES)

        def per_query(qi, total):
            q = qbase + qi
            qsplat = jnp.full((LANES,), q, jnp.int32)
            d0 = jnp.zeros((LANES,), jnp.int32)
            qx = plsc.load_gather(pc_v, [d0, qsplat])
            qy = plsc.load_gather(pc_v, [d0 + 1, qsplat])
            qz = plsc.load_gather(pc_v, [d0 + 2, qsplat])

            def scan_cond(carry):
                group, ptr = carry
                return jnp.logical_and(ptr < K_NB, group < ngroups)

            def scan_body(carry):
                group, ptr = carry
                gbase = group * (GROUP * LANES)
                ones = jnp.ones((LANES,), jnp.int32)

                @plsc.parallel_loop(0, GROUP, 1, unroll=GROUP,
                                    carry=jnp.full((LANES,), ptr - 1, jnp.int32))
                def run_loop(j, run_m1):
                    base = gbase + j * LANES
                    dx = pc_v[0, pl.ds(base, LANES)] - qx
                    dy = pc_v[1, pl.ds(base, LANES)] - qy
                    dz = pc_v[2, pl.ds(base, LANES)] - qz
                    d2 = dx * dx + dy * dy + dz * dz
                    m = d2 < R2
                    pos = run_m1 + plsc.cumsum(ones, mask=m)
                    plsc.store_scatter(idx_v, [pos], iota + base, mask=m)
                    return run_m1 + plsc.all_reduce_population_count(m)

                return group + 1, run_loop[0] + 1

            _, found = lax.while_loop(
                scan_cond, scan_body, (jnp.int32(0), jnp.int32(0)))

            cnt16 = jnp.minimum(jnp.maximum(found, 1), K_NB)
            idxv = idx_v[pl.ds(0, LANES)]
            valid = iota < cnt16
            first = idxv[0]
            idxall_v[pl.ds(qi * LANES, LANES)] = jnp.where(valid, idxv, first)
            return total

        lax.fori_loop(0, q_per_tile, per_query, jnp.int32(0))
        mask_copy.wait()

        def per_query_gather(qi, total):
            q = qbase + qi
            half = (C // 2) * N
            idx_c0 = idxall_v[pl.ds(qi * LANES, LANES)]
            q_c0 = jnp.full((LANES,), q, jnp.int32)
            idx_c1 = idx_c0 + half
            q_c1 = q_c0 + half
            acc0 = jnp.zeros((LANES,), jnp.float32)
            acc1 = jnp.zeros((LANES,), jnp.float32)
            for c in range(C // 2):
                nm0 = plsc.load_gather(mask_flat, [idx_c0])
                qm0 = plsc.load_gather(mask_flat, [q_c0])
                nm1 = plsc.load_gather(mask_flat, [idx_c1])
                qm1 = plsc.load_gather(mask_flat, [q_c1])
                acc0 = acc0 + jnp.abs(nm0 - qm0)
                acc1 = acc1 + jnp.abs(nm1 - qm1)
                if c + 1 < C // 2:
                    idx_c0 = idx_c0 + N
                    q_c0 = q_c0 + N
                    idx_c1 = idx_c1 + N
                    q_c1 = q_c1 + N
            return total + acc0 + acc1

        total = lax.fori_loop(0, q_per_tile, per_query_gather,
                              jnp.zeros((LANES,), jnp.float32))
        acc_v[...] = total
        pltpu.sync_copy(acc_v, out_hbm.at[pl.ds(wid * LANES, LANES)])

    return ballq


def kernel(pc, mask):
    B, N, _ = pc.shape
    C = mask.shape[-1]
    pcT = jnp.transpose(pc, (0, 2, 1))
    maskT = jnp.transpose(mask, (0, 2, 1)).reshape(B, C * N)
    partial = _build_sc_kernel(B, N, C)(pcT, maskT)
    return jnp.sum(partial) / (B * N * K_NB)
